# Initial kernel scaffold; baseline (speedup 1.0000x reference)
import jax
import jax.numpy as jnp
from jax.experimental import pallas as pl


def kernel(input_ids, char_positions, sentence_boundaries, char_pos_embed, seq_pos_embed, sentence_boundary_embed):
    raise NotImplementedError("write your pallas kernel here")


# speedup vs baseline: 17.8018x; 17.8018x over previous
import functools

import jax
import jax.numpy as jnp
from jax import lax
from jax.experimental import pallas as pl
from jax.experimental.pallas import tpu as pltpu
from jax.experimental.pallas import tpu_sc as plsc

B, S, H = 4096, 200, 128
NCP, NSB = 8, 3
TBL = S * NCP * NSB
NTOK = B * S
NW = 32
TOK_PER_W = NTOK // NW
CHUNK = 128
NCHUNK = TOK_PER_W // CHUNK
IDX_BLK = 512


def _table_body(char_ref, seq_ref, sbnd_ref, out_ref):
    row = lax.broadcasted_iota(jnp.int32, (TBL, S), 0)
    oh_s = (row // (NCP * NSB) == lax.broadcasted_iota(jnp.int32, (TBL, S), 1)).astype(jnp.float32)
    row_c = lax.broadcasted_iota(jnp.int32, (TBL, NCP), 0)
    oh_c = ((row_c // NSB) % NCP == lax.broadcasted_iota(jnp.int32, (TBL, NCP), 1)).astype(jnp.float32)
    row_k = lax.broadcasted_iota(jnp.int32, (TBL, NSB), 0)
    oh_k = (row_k % NSB == lax.broadcasted_iota(jnp.int32, (TBL, NSB), 1)).astype(jnp.float32)
    out_ref[...] = (
        jnp.dot(oh_s, seq_ref[...], preferred_element_type=jnp.float32)
        + jnp.dot(oh_c, char_ref[...], preferred_element_type=jnp.float32)
        + jnp.dot(oh_k, sbnd_ref[...], preferred_element_type=jnp.float32)
    )


def _idx_body(cp_ref, sb_ref, out_ref):
    s24 = lax.broadcasted_iota(jnp.int32, (IDX_BLK, S), 1) * (NCP * NSB)
    out_ref[...] = s24 + cp_ref[...] * NSB + sb_ref[...]


_sc_mesh = plsc.VectorSubcoreMesh(core_axis_name="c", subcore_axis_name="s")


@functools.partial(
    pl.kernel,
    mesh=_sc_mesh,
    out_type=jax.ShapeDtypeStruct((NTOK, H), jnp.float32),
    scratch_types=[
        pltpu.VMEM((NCHUNK, CHUNK), jnp.int32),
        pltpu.VMEM((CHUNK, H), jnp.float32),
        pltpu.SemaphoreType.DMA,
    ],
)
def _sc_gather(table_hbm, idx_hbm, out_hbm, idx_v, rows_v, gsem):
    wid = lax.axis_index("s") * 2 + lax.axis_index("c")
    base = wid * TOK_PER_W
    pltpu.sync_copy(idx_hbm.at[wid], idx_v)

    def step(j, carry):
        pltpu.async_copy(table_hbm.at[idx_v.at[j]], rows_v, gsem).wait()
        pltpu.sync_copy(rows_v, out_hbm.at[pl.ds(base + j * CHUNK, CHUNK)])
        return carry

    lax.fori_loop(0, NCHUNK, step, 0)


def kernel(input_ids, char_positions, sentence_boundaries, char_pos_embed, seq_pos_embed, sentence_boundary_embed):
    del input_ids
    table = pl.pallas_call(
        _table_body,
        out_shape=jax.ShapeDtypeStruct((TBL, H), jnp.float32),
    )(char_pos_embed, seq_pos_embed, sentence_boundary_embed)

    idx = pl.pallas_call(
        _idx_body,
        grid=(B // IDX_BLK,),
        in_specs=[
            pl.BlockSpec((IDX_BLK, S), lambda i: (i, 0)),
            pl.BlockSpec((IDX_BLK, S), lambda i: (i, 0)),
        ],
        out_specs=pl.BlockSpec((IDX_BLK, S), lambda i: (i, 0)),
        out_shape=jax.ShapeDtypeStruct((B, S), jnp.int32),
    )(char_positions.astype(jnp.int32), sentence_boundaries.astype(jnp.int32))

    idx3 = idx.reshape(NW, NCHUNK, CHUNK)
    out = _sc_gather(table, idx3)
    return out.reshape(B, S, H)


# speedup vs baseline: 40.7534x; 2.2893x over previous
import functools

import jax
import jax.numpy as jnp
from jax import lax
from jax.experimental import pallas as pl
from jax.experimental.pallas import tpu as pltpu
from jax.experimental.pallas import tpu_sc as plsc

B, S, H = 4096, 200, 128
NCP, NSB = 8, 3
TBL = S * NCP * NSB
NTOK = B * S
NW = 32
TOK_PER_W = NTOK // NW
CHUNK = 128
NCHUNK = TOK_PER_W // CHUNK
IDX_BLK = 512


def _table_body(char_ref, seq_ref, sbnd_ref, out_ref):
    row = lax.broadcasted_iota(jnp.int32, (TBL, S), 0)
    oh_s = (row // (NCP * NSB) == lax.broadcasted_iota(jnp.int32, (TBL, S), 1)).astype(jnp.float32)
    row_c = lax.broadcasted_iota(jnp.int32, (TBL, NCP), 0)
    oh_c = ((row_c // NSB) % NCP == lax.broadcasted_iota(jnp.int32, (TBL, NCP), 1)).astype(jnp.float32)
    row_k = lax.broadcasted_iota(jnp.int32, (TBL, NSB), 0)
    oh_k = (row_k % NSB == lax.broadcasted_iota(jnp.int32, (TBL, NSB), 1)).astype(jnp.float32)
    out_ref[...] = (
        jnp.dot(oh_s, seq_ref[...], preferred_element_type=jnp.float32)
        + jnp.dot(oh_c, char_ref[...], preferred_element_type=jnp.float32)
        + jnp.dot(oh_k, sbnd_ref[...], preferred_element_type=jnp.float32)
    )


def _idx_body(cp_ref, sb_ref, out_ref):
    s24 = lax.broadcasted_iota(jnp.int32, (IDX_BLK, S), 1) * (NCP * NSB)
    out_ref[...] = s24 + cp_ref[...] * NSB + sb_ref[...]


_sc_mesh = plsc.VectorSubcoreMesh(core_axis_name="c", subcore_axis_name="s")


@functools.partial(
    pl.kernel,
    mesh=_sc_mesh,
    out_type=jax.ShapeDtypeStruct((NTOK, H), jnp.float32),
    scratch_types=[
        pltpu.VMEM_SHARED((TBL, H), jnp.float32),
        pltpu.VMEM((NCHUNK, CHUNK), jnp.int32),
        pltpu.VMEM((CHUNK, H), jnp.float32),
        pltpu.VMEM((CHUNK, H), jnp.float32),
        pltpu.SemaphoreType.DMA,
        pltpu.SemaphoreType.DMA,
    ],
)
def _sc_gather(table_hbm, idx_hbm, out_hbm, table_sp, idx_v, buf0, buf1, sem0, sem1):
    sid = lax.axis_index("s")
    wid = sid * 2 + lax.axis_index("c")
    base = wid * TOK_PER_W

    @pl.when(sid == 0)
    def _():
        pltpu.sync_copy(table_hbm, table_sp)

    plsc.subcore_barrier()

    pltpu.sync_copy(idx_hbm.at[wid], idx_v)
    pltpu.async_copy(table_sp.at[idx_v.at[0]], buf0, sem0)

    def step(i, carry):
        j0 = 2 * i
        pltpu.make_async_copy(out_hbm.at[pl.ds(0, CHUNK)], buf0, sem0).wait()
        pltpu.async_copy(table_sp.at[idx_v.at[j0 + 1]], buf1, sem1)
        pltpu.sync_copy(buf0, out_hbm.at[pl.ds(base + j0 * CHUNK, CHUNK)])
        pltpu.make_async_copy(out_hbm.at[pl.ds(0, CHUNK)], buf1, sem1).wait()

        @pl.when(j0 + 2 < NCHUNK)
        def _():
            pltpu.async_copy(table_sp.at[idx_v.at[j0 + 2]], buf0, sem0)

        pltpu.sync_copy(buf1, out_hbm.at[pl.ds(base + (j0 + 1) * CHUNK, CHUNK)])
        return carry

    lax.fori_loop(0, NCHUNK // 2, step, 0)


def kernel(input_ids, char_positions, sentence_boundaries, char_pos_embed, seq_pos_embed, sentence_boundary_embed):
    del input_ids
    table = pl.pallas_call(
        _table_body,
        out_shape=jax.ShapeDtypeStruct((TBL, H), jnp.float32),
    )(char_pos_embed, seq_pos_embed, sentence_boundary_embed)

    idx = pl.pallas_call(
        _idx_body,
        grid=(B // IDX_BLK,),
        in_specs=[
            pl.BlockSpec((IDX_BLK, S), lambda i: (i, 0)),
            pl.BlockSpec((IDX_BLK, S), lambda i: (i, 0)),
        ],
        out_specs=pl.BlockSpec((IDX_BLK, S), lambda i: (i, 0)),
        out_shape=jax.ShapeDtypeStruct((B, S), jnp.int32),
    )(char_positions.astype(jnp.int32), sentence_boundaries.astype(jnp.int32))

    idx3 = idx.reshape(NW, NCHUNK, CHUNK)
    out = _sc_gather(table, idx3)
    return out.reshape(B, S, H)
